# baseline (speedup 1.0000x reference)
import functools

import jax
import jax.numpy as jnp
from jax import lax
from jax.experimental import pallas as pl
from jax.experimental.pallas import tpu as pltpu
from jax.experimental.pallas import tpu_sc as plsc

N_U = 5000
N_P = 5000
N = N_U + N_P
E = 320000
H = 128
EPS = 1e-5
HA = 144

NC = 2
NS = 16
NW = NC * NS
E2 = 2 * E
BE = 128
EP = 655360
NB = EP // (NW * BE)
CB = 16
AGG_ROWS = 10112
ZROWS = AGG_ROWS // NS
ROWS_OUT = ZROWS


def _seg_sum_call(feat, src_r, dst_r, zeros_feat, width):
  out_type = [jax.ShapeDtypeStruct((NC, AGG_ROWS, width), jnp.float32)]
  scratch = [
      pltpu.VMEM((CB, BE), jnp.int32),
      pltpu.VMEM((CB, BE), jnp.int32),
      pltpu.VMEM((BE, width), jnp.float32),
      pltpu.VMEM((BE, width), jnp.float32),
      pltpu.VMEM_SHARED((AGG_ROWS, width), jnp.float32),
      pltpu.SemaphoreType.DMA,
      pltpu.SemaphoreType.DMA,
      pltpu.SemaphoreType.DMA,
  ]

  mesh = plsc.VectorSubcoreMesh(core_axis_name="c", subcore_axis_name="s",
                                num_cores=NC, num_subcores=NS)

  def body(feat_h, src_h, dst_h, zf_h, out_agg, sidx, didx, rows0, rows1,
           agg, sem, sem_s0, sem_s1):
    c = lax.axis_index("c")
    s = lax.axis_index("s")
    w = c * NS + s
    bufs = (rows0, rows1)
    ssems = (sem_s0, sem_s1)

    pltpu.sync_copy(zf_h.at[pl.ds(s * ZROWS, ZROWS)],
                    agg.at[pl.ds(s * ZROWS, ZROWS)])
    plsc.subcore_barrier()

    def chunk(o, carry):
      pltpu.sync_copy(src_h.at[w, pl.ds(o * CB, CB)], sidx)
      pltpu.sync_copy(dst_h.at[w, pl.ds(o * CB, CB)], didx)
      scat = [None, None]
      d = pltpu.async_copy(feat_h.at[sidx.at[0]], bufs[0], sem)
      for j in range(CB):
        d.wait()
        if j + 1 < CB:
          if scat[(j + 1) % 2] is not None:
            scat[(j + 1) % 2].wait()
            scat[(j + 1) % 2] = None
          d = pltpu.async_copy(feat_h.at[sidx.at[j + 1]], bufs[(j + 1) % 2],
                               sem)
        scat[j % 2] = pltpu.async_copy(bufs[j % 2], agg.at[didx.at[j]],
                                       ssems[j % 2], add=True)
      for b in range(2):
        if scat[b] is not None:
          scat[b].wait()
      return carry

    lax.fori_loop(0, NB // CB, chunk, 0)
    plsc.subcore_barrier()

    pltpu.sync_copy(agg.at[pl.ds(s * ROWS_OUT, ROWS_OUT)],
                    out_agg.at[c, pl.ds(s * ROWS_OUT, ROWS_OUT)])

  fn = pl.kernel(body, out_type=out_type, mesh=mesh, scratch_types=scratch)
  return fn(feat, src_r, dst_r, zeros_feat)[0]


def _count_call(dst_r, zeros_feat, ones_rows):
  out_type = [jax.ShapeDtypeStruct((NC, AGG_ROWS, H), jnp.float32)]
  scratch = [
      pltpu.VMEM((CB, BE), jnp.int32),
      pltpu.VMEM((BE, H), jnp.float32),
      pltpu.VMEM_SHARED((AGG_ROWS, H), jnp.float32),
  ]
  mesh = plsc.VectorSubcoreMesh(core_axis_name="c", subcore_axis_name="s",
                                num_cores=NC, num_subcores=NS)

  def body(dst_h, zf_h, ones_h, out_agg, didx, ones, agg):
    c = lax.axis_index("c")
    s = lax.axis_index("s")
    w = c * NS + s
    pltpu.sync_copy(zf_h.at[pl.ds(s * ZROWS, ZROWS)],
                    agg.at[pl.ds(s * ZROWS, ZROWS)])
    pltpu.sync_copy(ones_h, ones)
    plsc.subcore_barrier()

    def chunk(o, carry):
      pltpu.sync_copy(dst_h.at[w, pl.ds(o * CB, CB)], didx)
      for j in range(CB):
        pltpu.sync_copy(ones, agg.at[didx.at[j]], add=True)
      return carry

    lax.fori_loop(0, NB // CB, chunk, 0)
    plsc.subcore_barrier()
    pltpu.sync_copy(agg.at[pl.ds(s * ROWS_OUT, ROWS_OUT)],
                    out_agg.at[c, pl.ds(s * ROWS_OUT, ROWS_OUT)])

  fn = pl.kernel(body, out_type=out_type, mesh=mesh, scratch_types=scratch)
  return fn(dst_r, zeros_feat, ones_rows)[0]


def _enc_body(x_ref, wt_ref, b_ref, o_ref):
  o_ref[...] = jnp.maximum(
      jnp.dot(x_ref[0], wt_ref[0], preferred_element_type=jnp.float32)
      + b_ref[0], 0.0)[None]


def _encode(xs, wts, bs):
  return pl.pallas_call(
      _enc_body,
      grid=(2,),
      in_specs=[
          pl.BlockSpec((1, N_U, H), lambda g: (g, 0, 0)),
          pl.BlockSpec((1, H, H), lambda g: (g, 0, 0)),
          pl.BlockSpec((1, 1, H), lambda g: (g, 0, 0)),
      ],
      out_specs=pl.BlockSpec((1, N_U, H), lambda g: (g, 0, 0)),
      out_shape=jax.ShapeDtypeStruct((2, N_U, H), jnp.float32),
  )(xs, wts, bs)


def _mid_body(aggA, aggB, cA, cB, x_ref, wlt, bl, wrt, gs, be, o_ref):
  cnt = cA[:, :1] + cB[:, :1]
  mean = (aggA[...] + aggB[...]) / jnp.maximum(cnt, 1.0)
  h = (jnp.dot(mean, wlt[...], preferred_element_type=jnp.float32) + bl[...]
       + jnp.dot(x_ref[...], wrt[...], preferred_element_type=jnp.float32))
  o_ref[...] = jnp.maximum(h * gs[...] + be[...], 0.0)


def _mid_layer(aggA, aggB, cA, cB, x, wlt, bl, wrt, gs, be):
  n = x.shape[0]
  args = (aggA, aggB, cA, cB, x, wlt, bl, wrt, gs, be)
  specs = [pl.BlockSpec(a.shape, lambda *_: tuple(0 for _ in a.shape))
           for a in args]
  return pl.pallas_call(
      _mid_body,
      in_specs=specs,
      out_specs=pl.BlockSpec((n, H), lambda *_: (0, 0)),
      out_shape=jax.ShapeDtypeStruct((n, H), jnp.float32),
  )(*args)


def _fin_body(aggA, aggB, cA, cB, h1, x0, wlt, bl, wrt, gs, be, wot, bo,
              o_ref):
  cnt = cA[:, :1] + cB[:, :1]
  mean = (aggA[...] + aggB[...]) / jnp.maximum(cnt, 1.0)
  h = (jnp.dot(mean, wlt[...], preferred_element_type=jnp.float32) + bl[...]
       + jnp.dot(h1[...], wrt[...], preferred_element_type=jnp.float32))
  h = jnp.maximum(h * gs[...] + be[...], 0.0) + x0[...]
  o_ref[...] = jnp.dot(h, wot[...], preferred_element_type=jnp.float32) + bo[...]


def _fin_layer(aggA, aggB, cA, cB, h1, x0, wlt, bl, wrt, gs, be, wot, bo):
  args = (aggA, aggB, cA, cB, h1, x0, wlt, bl, wrt, gs, be, wot, bo)
  specs = [pl.BlockSpec(a.shape, lambda *_: tuple(0 for _ in a.shape))
           for a in args]
  return pl.pallas_call(
      _fin_body,
      in_specs=specs,
      out_specs=pl.BlockSpec((N_U, H), lambda *_: (0, 0)),
      out_shape=jax.ShapeDtypeStruct((N_U, H), jnp.float32),
  )(*args)


def kernel(x_u, x_p, edge_index, W_u, b_u, W_p, b_p, W1_l, b1_l, W1_r, g1,
           be1, W2_l, b2_l, W2_r, g2, be2, W_out, b_out):
  s = 1.0 / jnp.sqrt(jnp.float32(1.0 + EPS))

  src = jnp.concatenate([edge_index[0], edge_index[1]]).astype(jnp.int32)
  dst = jnp.concatenate([edge_index[1], edge_index[0]]).astype(jnp.int32)
  pad = EP - E2
  src_r = jnp.concatenate([src, jnp.zeros((pad,), jnp.int32)]
                          ).reshape(NW, NB, BE)
  dst_r = jnp.concatenate([dst, jnp.full((pad,), N, jnp.int32)]
                          ).reshape(NW, NB, BE)
  zeros_feat = jnp.zeros((AGG_ROWS, H), jnp.float32)
  ones_rows = jnp.zeros((BE, H), jnp.float32).at[:, 0].set(1.0)

  xs = jnp.stack([x_u, x_p])
  wts = jnp.stack([W_u.T, W_p.T])
  bs = jnp.stack([b_u[None], b_p[None]])
  x0 = _encode(xs, wts, bs).reshape(N, H)

  cnt = _count_call(dst_r, zeros_feat, ones_rows)
  tok = (cnt[0, 0, 1] * 0.0).astype(jnp.int32)
  agg1 = _seg_sum_call(x0, src_r + tok, dst_r, zeros_feat, width=H)
  h1 = _mid_layer(agg1[0, :N], agg1[1, :N],
                  cnt[0, :N, :16], cnt[1, :N, :16], x0,
                  W1_l.T, b1_l[None], W1_r.T, (g1 * s)[None], be1[None])

  agg2 = _seg_sum_call(h1, src_r, dst_r, zeros_feat, width=H)
  wot = jnp.zeros((H, H), jnp.float32).at[:, 0].set(W_out[0])
  bo = jnp.zeros((1, H), jnp.float32).at[0, 0].set(b_out[0])
  out_full = _fin_layer(agg2[0, :N_U], agg2[1, :N_U],
                        cnt[0, :N_U, :16], cnt[1, :N_U, :16],
                        h1[:N_U], x0[:N_U],
                        W2_l.T, b2_l[None], W2_r.T, (g2 * s)[None],
                        be2[None], wot, bo)
  return out_full[:, :1]


# speedup vs baseline: 1.0871x; 1.0871x over previous
import functools

import jax
import jax.numpy as jnp
from jax import lax
from jax.experimental import pallas as pl
from jax.experimental.pallas import tpu as pltpu
from jax.experimental.pallas import tpu_sc as plsc

N_U = 5000
N_P = 5000
N = N_U + N_P
E = 320000
H = 128
EPS = 1e-5
HA = 144

NC = 2
NS = 16
NW = NC * NS
E2 = 2 * E
BE = 64
EP = 655360
NB = EP // (NW * BE)
CB = 32
NBUF = 4
DEPTH = 3
AGG_ROWS = 10112
ZROWS = AGG_ROWS // NS
ROWS_OUT = ZROWS


def _seg_sum_call(feat, src_r, dst_r, zeros_feat, width):
  out_type = [jax.ShapeDtypeStruct((NC, AGG_ROWS, width), jnp.float32)]
  scratch = (
      [pltpu.VMEM((CB, BE), jnp.int32),
       pltpu.VMEM((CB, BE), jnp.int32)]
      + [pltpu.VMEM((BE, width), jnp.float32) for _ in range(NBUF)]
      + [pltpu.VMEM_SHARED((AGG_ROWS, width), jnp.float32)]
      + [pltpu.SemaphoreType.DMA for _ in range(2 * NBUF)]
  )

  mesh = plsc.VectorSubcoreMesh(core_axis_name="c", subcore_axis_name="s",
                                num_cores=NC, num_subcores=NS)

  def body(feat_h, src_h, dst_h, zf_h, out_agg, sidx, didx, *rest):
    bufs = rest[:NBUF]
    agg = rest[NBUF]
    gsems = rest[NBUF + 1:NBUF + 1 + NBUF]
    ssems = rest[NBUF + 1 + NBUF:]
    c = lax.axis_index("c")
    s = lax.axis_index("s")
    w = c * NS + s

    pltpu.sync_copy(zf_h.at[pl.ds(s * ZROWS, ZROWS)],
                    agg.at[pl.ds(s * ZROWS, ZROWS)])
    plsc.subcore_barrier()

    def chunk(o, carry):
      pltpu.sync_copy(src_h.at[w, pl.ds(o * CB, CB)], sidx)
      pltpu.sync_copy(dst_h.at[w, pl.ds(o * CB, CB)], didx)
      gd = [None] * NBUF
      sd = [None] * NBUF
      for j in range(DEPTH):
        gd[j] = pltpu.async_copy(feat_h.at[sidx.at[j]], bufs[j], gsems[j])
      for j in range(CB):
        b = j % NBUF
        gd[b].wait()
        gd[b] = None
        jj = j + DEPTH
        if jj < CB:
          bb = jj % NBUF
          if sd[bb] is not None:
            sd[bb].wait()
            sd[bb] = None
          gd[bb] = pltpu.async_copy(feat_h.at[sidx.at[jj]], bufs[bb],
                                    gsems[bb])
        sd[b] = pltpu.async_copy(bufs[b], agg.at[didx.at[j]], ssems[b],
                                 add=True)
      for b in range(NBUF):
        if sd[b] is not None:
          sd[b].wait()
      return carry

    lax.fori_loop(0, NB // CB, chunk, 0)
    plsc.subcore_barrier()

    pltpu.sync_copy(agg.at[pl.ds(s * ROWS_OUT, ROWS_OUT)],
                    out_agg.at[c, pl.ds(s * ROWS_OUT, ROWS_OUT)])

  fn = pl.kernel(body, out_type=out_type, mesh=mesh, scratch_types=scratch)
  return fn(feat, src_r, dst_r, zeros_feat)[0]


def _count_call(dst_r, zeros_feat, ones_rows):
  out_type = [jax.ShapeDtypeStruct((NC, AGG_ROWS, H), jnp.float32)]
  scratch = [
      pltpu.VMEM((CB, BE), jnp.int32),
      pltpu.VMEM((BE, H), jnp.float32),
      pltpu.VMEM_SHARED((AGG_ROWS, H), jnp.float32),
  ]
  mesh = plsc.VectorSubcoreMesh(core_axis_name="c", subcore_axis_name="s",
                                num_cores=NC, num_subcores=NS)

  def body(dst_h, zf_h, ones_h, out_agg, didx, ones, agg):
    c = lax.axis_index("c")
    s = lax.axis_index("s")
    w = c * NS + s
    pltpu.sync_copy(zf_h.at[pl.ds(s * ZROWS, ZROWS)],
                    agg.at[pl.ds(s * ZROWS, ZROWS)])
    pltpu.sync_copy(ones_h, ones)
    plsc.subcore_barrier()

    def chunk(o, carry):
      pltpu.sync_copy(dst_h.at[w, pl.ds(o * CB, CB)], didx)
      for j in range(CB):
        pltpu.sync_copy(ones, agg.at[didx.at[j]], add=True)
      return carry

    lax.fori_loop(0, NB // CB, chunk, 0)
    plsc.subcore_barrier()
    pltpu.sync_copy(agg.at[pl.ds(s * ROWS_OUT, ROWS_OUT)],
                    out_agg.at[c, pl.ds(s * ROWS_OUT, ROWS_OUT)])

  fn = pl.kernel(body, out_type=out_type, mesh=mesh, scratch_types=scratch)
  return fn(dst_r, zeros_feat, ones_rows)[0]


def _enc_body(x_ref, wt_ref, b_ref, o_ref):
  o_ref[...] = jnp.maximum(
      jnp.dot(x_ref[0], wt_ref[0], preferred_element_type=jnp.float32)
      + b_ref[0], 0.0)[None]


def _encode(xs, wts, bs):
  return pl.pallas_call(
      _enc_body,
      grid=(2,),
      in_specs=[
          pl.BlockSpec((1, N_U, H), lambda g: (g, 0, 0)),
          pl.BlockSpec((1, H, H), lambda g: (g, 0, 0)),
          pl.BlockSpec((1, 1, H), lambda g: (g, 0, 0)),
      ],
      out_specs=pl.BlockSpec((1, N_U, H), lambda g: (g, 0, 0)),
      out_shape=jax.ShapeDtypeStruct((2, N_U, H), jnp.float32),
  )(xs, wts, bs)


def _mid_body(aggA, aggB, cA, cB, x_ref, wlt, bl, wrt, gs, be, o_ref):
  cnt = cA[:, :1] + cB[:, :1]
  mean = (aggA[...] + aggB[...]) / jnp.maximum(cnt, 1.0)
  h = (jnp.dot(mean, wlt[...], preferred_element_type=jnp.float32) + bl[...]
       + jnp.dot(x_ref[...], wrt[...], preferred_element_type=jnp.float32))
  o_ref[...] = jnp.maximum(h * gs[...] + be[...], 0.0)


def _mid_layer(aggA, aggB, cA, cB, x, wlt, bl, wrt, gs, be):
  n = x.shape[0]
  args = (aggA, aggB, cA, cB, x, wlt, bl, wrt, gs, be)
  specs = [pl.BlockSpec(a.shape, lambda *_: tuple(0 for _ in a.shape))
           for a in args]
  return pl.pallas_call(
      _mid_body,
      in_specs=specs,
      out_specs=pl.BlockSpec((n, H), lambda *_: (0, 0)),
      out_shape=jax.ShapeDtypeStruct((n, H), jnp.float32),
  )(*args)


def _fin_body(aggA, aggB, cA, cB, h1, x0, wlt, bl, wrt, gs, be, wot, bo,
              o_ref):
  cnt = cA[:, :1] + cB[:, :1]
  mean = (aggA[...] + aggB[...]) / jnp.maximum(cnt, 1.0)
  h = (jnp.dot(mean, wlt[...], preferred_element_type=jnp.float32) + bl[...]
       + jnp.dot(h1[...], wrt[...], preferred_element_type=jnp.float32))
  h = jnp.maximum(h * gs[...] + be[...], 0.0) + x0[...]
  o_ref[...] = jnp.dot(h, wot[...], preferred_element_type=jnp.float32) + bo[...]


def _fin_layer(aggA, aggB, cA, cB, h1, x0, wlt, bl, wrt, gs, be, wot, bo):
  args = (aggA, aggB, cA, cB, h1, x0, wlt, bl, wrt, gs, be, wot, bo)
  specs = [pl.BlockSpec(a.shape, lambda *_: tuple(0 for _ in a.shape))
           for a in args]
  return pl.pallas_call(
      _fin_body,
      in_specs=specs,
      out_specs=pl.BlockSpec((N_U, H), lambda *_: (0, 0)),
      out_shape=jax.ShapeDtypeStruct((N_U, H), jnp.float32),
  )(*args)


def kernel(x_u, x_p, edge_index, W_u, b_u, W_p, b_p, W1_l, b1_l, W1_r, g1,
           be1, W2_l, b2_l, W2_r, g2, be2, W_out, b_out):
  s = 1.0 / jnp.sqrt(jnp.float32(1.0 + EPS))

  src = jnp.concatenate([edge_index[0], edge_index[1]]).astype(jnp.int32)
  dst = jnp.concatenate([edge_index[1], edge_index[0]]).astype(jnp.int32)
  pad = EP - E2
  src_r = jnp.concatenate([src, jnp.zeros((pad,), jnp.int32)]
                          ).reshape(NW, NB, BE)
  dst_r = jnp.concatenate([dst, jnp.full((pad,), N, jnp.int32)]
                          ).reshape(NW, NB, BE)
  zeros_feat = jnp.zeros((AGG_ROWS, H), jnp.float32)
  ones_rows = jnp.zeros((BE, H), jnp.float32).at[:, 0].set(1.0)

  xs = jnp.stack([x_u, x_p])
  wts = jnp.stack([W_u.T, W_p.T])
  bs = jnp.stack([b_u[None], b_p[None]])
  x0 = _encode(xs, wts, bs).reshape(N, H)

  cnt = _count_call(dst_r, zeros_feat, ones_rows)
  tok = (cnt[0, 0, 1] * 0.0).astype(jnp.int32)
  agg1 = _seg_sum_call(x0, src_r + tok, dst_r, zeros_feat, width=H)
  h1 = _mid_layer(agg1[0, :N], agg1[1, :N],
                  cnt[0, :N, :16], cnt[1, :N, :16], x0,
                  W1_l.T, b1_l[None], W1_r.T, (g1 * s)[None], be1[None])

  agg2 = _seg_sum_call(h1, src_r, dst_r, zeros_feat, width=H)
  wot = jnp.zeros((H, H), jnp.float32).at[:, 0].set(W_out[0])
  bo = jnp.zeros((1, H), jnp.float32).at[0, 0].set(b_out[0])
  out_full = _fin_layer(agg2[0, :N_U], agg2[1, :N_U],
                        cnt[0, :N_U, :16], cnt[1, :N_U, :16],
                        h1[:N_U], x0[:N_U],
                        W2_l.T, b2_l[None], W2_r.T, (g2 * s)[None],
                        be2[None], wot, bo)
  return out_full[:, :1]


# speedup vs baseline: 2.6505x; 2.4383x over previous
import functools

import jax
import jax.numpy as jnp
from jax import lax
from jax.experimental import pallas as pl
from jax.experimental.pallas import tpu as pltpu
from jax.experimental.pallas import tpu_sc as plsc

N_U = 5000
N_P = 5000
N = N_U + N_P
E = 320000
H = 128
EPS = 1e-5
HA = 144

NC = 2
NS = 16
NW = NC * NS
E2 = 2 * E
BE = 64
EP = 655360
NB = EP // (NW * BE)
CB = 32
NBUF = 4
DEPTH = 3
HW2 = 64
NB2 = EP // (NS * BE)
AGG_ROWS = 10112
ZROWS = AGG_ROWS // NS
ROWS_OUT = ZROWS


def _seg_sum_call(feat_split, src_r2, dst_r2, zeros_half):
  out_type = [jax.ShapeDtypeStruct((NC, AGG_ROWS, HW2), jnp.float32)]
  scratch = (
      [pltpu.VMEM((CB, BE), jnp.int32),
       pltpu.VMEM((CB, BE), jnp.int32)]
      + [pltpu.VMEM((BE, HW2), jnp.float32) for _ in range(NBUF)]
      + [pltpu.VMEM_SHARED((AGG_ROWS, HW2), jnp.float32),
         pltpu.VMEM_SHARED((AGG_ROWS, HW2), jnp.float32)]
      + [pltpu.SemaphoreType.DMA for _ in range(2 * NBUF)]
  )

  mesh = plsc.VectorSubcoreMesh(core_axis_name="c", subcore_axis_name="s",
                                num_cores=NC, num_subcores=NS)

  def body(fs_h, src_h, dst_h, zf_h, out_agg, sidx, didx, *rest):
    bufs = rest[:NBUF]
    xs = rest[NBUF]
    agg = rest[NBUF + 1]
    gsems = rest[NBUF + 2:NBUF + 2 + NBUF]
    ssems = rest[NBUF + 2 + NBUF:]
    c = lax.axis_index("c")
    s = lax.axis_index("s")

    pltpu.sync_copy(fs_h.at[c, pl.ds(s * ZROWS, ZROWS)],
                    xs.at[pl.ds(s * ZROWS, ZROWS)])
    pltpu.sync_copy(zf_h.at[pl.ds(s * ZROWS, ZROWS)],
                    agg.at[pl.ds(s * ZROWS, ZROWS)])
    plsc.subcore_barrier()

    def chunk(o, carry):
      pltpu.sync_copy(src_h.at[s, pl.ds(o * CB, CB)], sidx)
      pltpu.sync_copy(dst_h.at[s, pl.ds(o * CB, CB)], didx)
      gd = [None] * NBUF
      sd = [None] * NBUF
      for j in range(DEPTH):
        gd[j] = pltpu.async_copy(xs.at[sidx.at[j]], bufs[j], gsems[j])
      for j in range(CB):
        b = j % NBUF
        gd[b].wait()
        gd[b] = None
        jj = j + DEPTH
        if jj < CB:
          bb = jj % NBUF
          if sd[bb] is not None:
            sd[bb].wait()
            sd[bb] = None
          gd[bb] = pltpu.async_copy(xs.at[sidx.at[jj]], bufs[bb],
                                    gsems[bb])
        sd[b] = pltpu.async_copy(bufs[b], agg.at[didx.at[j]], ssems[b],
                                 add=True)
      for b in range(NBUF):
        if sd[b] is not None:
          sd[b].wait()
      return carry

    lax.fori_loop(0, NB2 // CB, chunk, 0)
    plsc.subcore_barrier()

    pltpu.sync_copy(agg.at[pl.ds(s * ROWS_OUT, ROWS_OUT)],
                    out_agg.at[c, pl.ds(s * ROWS_OUT, ROWS_OUT)])

  fn = pl.kernel(body, out_type=out_type, mesh=mesh, scratch_types=scratch,
                 compiler_params=pltpu.CompilerParams(
                     use_tc_tiling_on_sc=False))
  return fn(feat_split, src_r2, dst_r2, zeros_half)[0]


def _count_call(dst_r, zeros_feat, ones_rows):
  out_type = [jax.ShapeDtypeStruct((NC, AGG_ROWS, H), jnp.float32)]
  scratch = [
      pltpu.VMEM((CB, BE), jnp.int32),
      pltpu.VMEM((BE, H), jnp.float32),
      pltpu.VMEM_SHARED((AGG_ROWS, H), jnp.float32),
  ]
  mesh = plsc.VectorSubcoreMesh(core_axis_name="c", subcore_axis_name="s",
                                num_cores=NC, num_subcores=NS)

  def body(dst_h, zf_h, ones_h, out_agg, didx, ones, agg):
    c = lax.axis_index("c")
    s = lax.axis_index("s")
    w = c * NS + s
    pltpu.sync_copy(zf_h.at[pl.ds(s * ZROWS, ZROWS)],
                    agg.at[pl.ds(s * ZROWS, ZROWS)])
    pltpu.sync_copy(ones_h, ones)
    plsc.subcore_barrier()

    def chunk(o, carry):
      pltpu.sync_copy(dst_h.at[w, pl.ds(o * CB, CB)], didx)
      for j in range(CB):
        pltpu.sync_copy(ones, agg.at[didx.at[j]], add=True)
      return carry

    lax.fori_loop(0, NB // CB, chunk, 0)
    plsc.subcore_barrier()
    pltpu.sync_copy(agg.at[pl.ds(s * ROWS_OUT, ROWS_OUT)],
                    out_agg.at[c, pl.ds(s * ROWS_OUT, ROWS_OUT)])

  fn = pl.kernel(body, out_type=out_type, mesh=mesh, scratch_types=scratch)
  return fn(dst_r, zeros_feat, ones_rows)[0]


def _enc_body(x_ref, wt_ref, b_ref, o_ref):
  o_ref[...] = jnp.maximum(
      jnp.dot(x_ref[0], wt_ref[0], preferred_element_type=jnp.float32)
      + b_ref[0], 0.0)[None]


def _encode(xs, wts, bs):
  return pl.pallas_call(
      _enc_body,
      grid=(2,),
      in_specs=[
          pl.BlockSpec((1, N_U, H), lambda g: (g, 0, 0)),
          pl.BlockSpec((1, H, H), lambda g: (g, 0, 0)),
          pl.BlockSpec((1, 1, H), lambda g: (g, 0, 0)),
      ],
      out_specs=pl.BlockSpec((1, N_U, H), lambda g: (g, 0, 0)),
      out_shape=jax.ShapeDtypeStruct((2, N_U, H), jnp.float32),
  )(xs, wts, bs)


def _mid_body(agg, cA, cB, x_ref, wlt, bl, wrt, gs, be, o_ref):
  cnt = cA[:, :1] + cB[:, :1]
  mean = agg[...] / jnp.maximum(cnt, 1.0)
  h = (jnp.dot(mean, wlt[...], preferred_element_type=jnp.float32) + bl[...]
       + jnp.dot(x_ref[...], wrt[...], preferred_element_type=jnp.float32))
  o_ref[...] = jnp.maximum(h * gs[...] + be[...], 0.0)


def _mid_layer(agg, cA, cB, x, wlt, bl, wrt, gs, be):
  n = x.shape[0]
  args = (agg, cA, cB, x, wlt, bl, wrt, gs, be)
  specs = [pl.BlockSpec(a.shape, lambda *_: tuple(0 for _ in a.shape))
           for a in args]
  return pl.pallas_call(
      _mid_body,
      in_specs=specs,
      out_specs=pl.BlockSpec((n, H), lambda *_: (0, 0)),
      out_shape=jax.ShapeDtypeStruct((n, H), jnp.float32),
  )(*args)


def _fin_body(agg, cA, cB, h1, x0, wlt, bl, wrt, gs, be, wot, bo,
              o_ref):
  cnt = cA[:, :1] + cB[:, :1]
  mean = agg[...] / jnp.maximum(cnt, 1.0)
  h = (jnp.dot(mean, wlt[...], preferred_element_type=jnp.float32) + bl[...]
       + jnp.dot(h1[...], wrt[...], preferred_element_type=jnp.float32))
  h = jnp.maximum(h * gs[...] + be[...], 0.0) + x0[...]
  o_ref[...] = jnp.dot(h, wot[...], preferred_element_type=jnp.float32) + bo[...]


def _fin_layer(agg, cA, cB, h1, x0, wlt, bl, wrt, gs, be, wot, bo):
  args = (agg, cA, cB, h1, x0, wlt, bl, wrt, gs, be, wot, bo)
  specs = [pl.BlockSpec(a.shape, lambda *_: tuple(0 for _ in a.shape))
           for a in args]
  return pl.pallas_call(
      _fin_body,
      in_specs=specs,
      out_specs=pl.BlockSpec((N_U, H), lambda *_: (0, 0)),
      out_shape=jax.ShapeDtypeStruct((N_U, H), jnp.float32),
  )(*args)


def _split_cols(feat):
  fp = jnp.pad(feat, ((0, AGG_ROWS - N), (0, 0)))
  return fp.reshape(AGG_ROWS, NC, HW2).transpose(1, 0, 2)


def kernel(x_u, x_p, edge_index, W_u, b_u, W_p, b_p, W1_l, b1_l, W1_r, g1,
           be1, W2_l, b2_l, W2_r, g2, be2, W_out, b_out):
  s = 1.0 / jnp.sqrt(jnp.float32(1.0 + EPS))

  src = jnp.concatenate([edge_index[0], edge_index[1]]).astype(jnp.int32)
  dst = jnp.concatenate([edge_index[1], edge_index[0]]).astype(jnp.int32)
  pad = EP - E2
  src_p = jnp.concatenate([src, jnp.zeros((pad,), jnp.int32)])
  dst_p = jnp.concatenate([dst, jnp.full((pad,), N, jnp.int32)])
  dst_r = dst_p.reshape(NW, NB, BE)
  src_r2 = src_p.reshape(NS, NB2, BE)
  dst_r2 = dst_p.reshape(NS, NB2, BE)
  zeros_feat = jnp.zeros((AGG_ROWS, H), jnp.float32)
  zeros_half = jnp.zeros((AGG_ROWS, HW2), jnp.float32)
  ones_rows = jnp.zeros((BE, H), jnp.float32).at[:, 0].set(1.0)

  xs = jnp.stack([x_u, x_p])
  wts = jnp.stack([W_u.T, W_p.T])
  bs = jnp.stack([b_u[None], b_p[None]])
  x0 = _encode(xs, wts, bs).reshape(N, H)

  cnt = _count_call(dst_r, zeros_feat, ones_rows)
  tok = (cnt[0, 0, 1] * 0.0).astype(jnp.int32)
  agg1 = _seg_sum_call(_split_cols(x0), src_r2 + tok, dst_r2, zeros_half)
  agg1f = jnp.concatenate([agg1[0, :N], agg1[1, :N]], axis=1)
  h1 = _mid_layer(agg1f, cnt[0, :N, :16], cnt[1, :N, :16], x0,
                  W1_l.T, b1_l[None], W1_r.T, (g1 * s)[None], be1[None])

  agg2 = _seg_sum_call(_split_cols(h1), src_r2, dst_r2, zeros_half)
  agg2f = jnp.concatenate([agg2[0, :N_U], agg2[1, :N_U]], axis=1)
  wot = jnp.zeros((H, H), jnp.float32).at[:, 0].set(W_out[0])
  bo = jnp.zeros((1, H), jnp.float32).at[0, 0].set(b_out[0])
  out_full = _fin_layer(agg2f, cnt[0, :N_U, :16], cnt[1, :N_U, :16],
                        h1[:N_U], x0[:N_U],
                        W2_l.T, b2_l[None], W2_r.T, (g2 * s)[None],
                        be2[None], wot, bo)
  return out_full[:, :1]


# speedup vs baseline: 2.7244x; 1.0279x over previous
import functools

import jax
import jax.numpy as jnp
from jax import lax
from jax.experimental import pallas as pl
from jax.experimental.pallas import tpu as pltpu
from jax.experimental.pallas import tpu_sc as plsc

N_U = 5000
N_P = 5000
N = N_U + N_P
E = 320000
H = 128
EPS = 1e-5
HA = 144

NC = 2
NS = 16
NW = NC * NS
E2 = 2 * E
BE = 64
EP = 655360
NB = EP // (NW * BE)
CB = 32
NBUF = 4
DEPTH = 3
HW2 = 64
HC = 80
NB2 = EP // (NS * BE)
AGG_ROWS = 10112
ZROWS = AGG_ROWS // NS
ROWS_OUT = ZROWS


def _seg_sum_call(feat_split, src_r2, dst_r2, zeros_half, width):
  out_type = [jax.ShapeDtypeStruct((NC, AGG_ROWS, width), jnp.float32)]
  scratch = (
      [pltpu.VMEM((CB, BE), jnp.int32),
       pltpu.VMEM((CB, BE), jnp.int32)]
      + [pltpu.VMEM((BE, width), jnp.float32) for _ in range(NBUF)]
      + [pltpu.VMEM_SHARED((AGG_ROWS, width), jnp.float32),
         pltpu.VMEM_SHARED((AGG_ROWS, width), jnp.float32)]
      + [pltpu.SemaphoreType.DMA for _ in range(2 * NBUF)]
  )

  mesh = plsc.VectorSubcoreMesh(core_axis_name="c", subcore_axis_name="s",
                                num_cores=NC, num_subcores=NS)

  def body(fs_h, src_h, dst_h, zf_h, out_agg, sidx, didx, *rest):
    bufs = rest[:NBUF]
    xs = rest[NBUF]
    agg = rest[NBUF + 1]
    gsems = rest[NBUF + 2:NBUF + 2 + NBUF]
    ssems = rest[NBUF + 2 + NBUF:]
    c = lax.axis_index("c")
    s = lax.axis_index("s")

    pltpu.sync_copy(fs_h.at[c, pl.ds(s * ZROWS, ZROWS)],
                    xs.at[pl.ds(s * ZROWS, ZROWS)])
    pltpu.sync_copy(zf_h.at[pl.ds(s * ZROWS, ZROWS)],
                    agg.at[pl.ds(s * ZROWS, ZROWS)])
    plsc.subcore_barrier()

    def chunk(o, carry):
      pltpu.sync_copy(src_h.at[s, pl.ds(o * CB, CB)], sidx)
      pltpu.sync_copy(dst_h.at[s, pl.ds(o * CB, CB)], didx)
      gd = [None] * NBUF
      sd = [None] * NBUF
      for j in range(DEPTH):
        gd[j] = pltpu.async_copy(xs.at[sidx.at[j]], bufs[j], gsems[j])
      for j in range(CB):
        b = j % NBUF
        gd[b].wait()
        gd[b] = None
        jj = j + DEPTH
        if jj < CB:
          bb = jj % NBUF
          if sd[bb] is not None:
            sd[bb].wait()
            sd[bb] = None
          gd[bb] = pltpu.async_copy(xs.at[sidx.at[jj]], bufs[bb],
                                    gsems[bb])
        sd[b] = pltpu.async_copy(bufs[b], agg.at[didx.at[j]], ssems[b],
                                 add=True)
      for b in range(NBUF):
        if sd[b] is not None:
          sd[b].wait()
      return carry

    lax.fori_loop(0, NB2 // CB, chunk, 0)
    plsc.subcore_barrier()

    pltpu.sync_copy(agg.at[pl.ds(s * ROWS_OUT, ROWS_OUT)],
                    out_agg.at[c, pl.ds(s * ROWS_OUT, ROWS_OUT)])

  fn = pl.kernel(body, out_type=out_type, mesh=mesh, scratch_types=scratch,
                 compiler_params=pltpu.CompilerParams(
                     use_tc_tiling_on_sc=False))
  return fn(feat_split, src_r2, dst_r2, zeros_half)[0]


def _enc_body(x_ref, wt_ref, b_ref, o_ref):
  o_ref[...] = jnp.maximum(
      jnp.dot(x_ref[0], wt_ref[0], preferred_element_type=jnp.float32)
      + b_ref[0], 0.0)[None]


def _encode(xs, wts, bs):
  return pl.pallas_call(
      _enc_body,
      grid=(2,),
      in_specs=[
          pl.BlockSpec((1, N_U, H), lambda g: (g, 0, 0)),
          pl.BlockSpec((1, H, H), lambda g: (g, 0, 0)),
          pl.BlockSpec((1, 1, H), lambda g: (g, 0, 0)),
      ],
      out_specs=pl.BlockSpec((1, N_U, H), lambda g: (g, 0, 0)),
      out_shape=jax.ShapeDtypeStruct((2, N_U, H), jnp.float32),
  )(xs, wts, bs)


def _mid_body(agg, cnt16, x_ref, wlt, bl, wrt, gs, be, o_ref):
  cnt = cnt16[:, :1]
  mean = agg[...] / jnp.maximum(cnt, 1.0)
  h = (jnp.dot(mean, wlt[...], preferred_element_type=jnp.float32) + bl[...]
       + jnp.dot(x_ref[...], wrt[...], preferred_element_type=jnp.float32))
  o_ref[...] = jnp.maximum(h * gs[...] + be[...], 0.0)


def _mid_layer(agg, cnt16, x, wlt, bl, wrt, gs, be):
  n = x.shape[0]
  args = (agg, cnt16, x, wlt, bl, wrt, gs, be)
  specs = [pl.BlockSpec(a.shape, lambda *_: tuple(0 for _ in a.shape))
           for a in args]
  return pl.pallas_call(
      _mid_body,
      in_specs=specs,
      out_specs=pl.BlockSpec((n, H), lambda *_: (0, 0)),
      out_shape=jax.ShapeDtypeStruct((n, H), jnp.float32),
  )(*args)


def _fin_body(agg, cnt16, h1, x0, wlt, bl, wrt, gs, be, wot, bo,
              o_ref):
  cnt = cnt16[:, :1]
  mean = agg[...] / jnp.maximum(cnt, 1.0)
  h = (jnp.dot(mean, wlt[...], preferred_element_type=jnp.float32) + bl[...]
       + jnp.dot(h1[...], wrt[...], preferred_element_type=jnp.float32))
  h = jnp.maximum(h * gs[...] + be[...], 0.0) + x0[...]
  o_ref[...] = jnp.dot(h, wot[...], preferred_element_type=jnp.float32) + bo[...]


def _fin_layer(agg, cnt16, h1, x0, wlt, bl, wrt, gs, be, wot, bo):
  args = (agg, cnt16, h1, x0, wlt, bl, wrt, gs, be, wot, bo)
  specs = [pl.BlockSpec(a.shape, lambda *_: tuple(0 for _ in a.shape))
           for a in args]
  return pl.pallas_call(
      _fin_body,
      in_specs=specs,
      out_specs=pl.BlockSpec((N_U, H), lambda *_: (0, 0)),
      out_shape=jax.ShapeDtypeStruct((N_U, H), jnp.float32),
  )(*args)


def _split_cols(feat, with_ones):
  fp = jnp.pad(feat, ((0, AGG_ROWS - N), (0, 0)))
  halves = fp.reshape(AGG_ROWS, NC, HW2).transpose(1, 0, 2)
  if not with_ones:
    return halves
  ones = (jnp.arange(AGG_ROWS) < N).astype(jnp.float32)[None, :, None]
  ones = jnp.broadcast_to(ones, (NC, AGG_ROWS, 1))
  zpad = jnp.zeros((NC, AGG_ROWS, HC - HW2 - 1), jnp.float32)
  return jnp.concatenate([halves, ones, zpad], axis=2)


def kernel(x_u, x_p, edge_index, W_u, b_u, W_p, b_p, W1_l, b1_l, W1_r, g1,
           be1, W2_l, b2_l, W2_r, g2, be2, W_out, b_out):
  s = 1.0 / jnp.sqrt(jnp.float32(1.0 + EPS))

  src = jnp.concatenate([edge_index[0], edge_index[1]]).astype(jnp.int32)
  dst = jnp.concatenate([edge_index[1], edge_index[0]]).astype(jnp.int32)
  pad = EP - E2
  src_p = jnp.concatenate([src, jnp.zeros((pad,), jnp.int32)])
  dst_p = jnp.concatenate([dst, jnp.full((pad,), N, jnp.int32)])
  src_r2 = src_p.reshape(NS, NB2, BE)
  dst_r2 = dst_p.reshape(NS, NB2, BE)
  zeros_cnt = jnp.zeros((AGG_ROWS, HC), jnp.float32)
  zeros_half = jnp.zeros((AGG_ROWS, HW2), jnp.float32)

  xs = jnp.stack([x_u, x_p])
  wts = jnp.stack([W_u.T, W_p.T])
  bs = jnp.stack([b_u[None], b_p[None]])
  x0 = _encode(xs, wts, bs).reshape(N, H)

  agg1 = _seg_sum_call(_split_cols(x0, True), src_r2, dst_r2, zeros_cnt,
                       width=HC)
  agg1f = jnp.concatenate([agg1[0, :N, :HW2], agg1[1, :N, :HW2]], axis=1)
  cnt16 = agg1[0, :, HW2:HW2 + 16]
  h1 = _mid_layer(agg1f, cnt16[:N], x0,
                  W1_l.T, b1_l[None], W1_r.T, (g1 * s)[None], be1[None])

  agg2 = _seg_sum_call(_split_cols(h1, False), src_r2, dst_r2, zeros_half,
                       width=HW2)
  agg2f = jnp.concatenate([agg2[0, :N_U], agg2[1, :N_U]], axis=1)
  wot = jnp.zeros((H, H), jnp.float32).at[:, 0].set(W_out[0])
  bo = jnp.zeros((1, H), jnp.float32).at[0, 0].set(b_out[0])
  out_full = _fin_layer(agg2f, cnt16[:N_U], h1[:N_U], x0[:N_U],
                        W2_l.T, b2_l[None], W2_r.T, (g2 * s)[None],
                        be2[None], wot, bo)
  return out_full[:, :1]
